# baseline (device time: 19791 ns/iter reference)
import os

import jax
import jax.numpy as jnp
from jax import lax
from jax.experimental import pallas as pl
from jax.experimental.pallas import tpu as pltpu

N_DEV = 4
N_LAYERS = 3
NO_COMM = os.path.exists(os.path.join(os.path.dirname(__file__), "NO_COMM"))


def kernel(x, Win0, Wout0, Win1, Wout1, Win2, Wout2):
    B, D = x.shape
    H = Win0.shape[1]
    rows_per = B // N_DEV

    def body(x_hbm, win0_hbm, wout0_hbm, win1_hbm, wout1_hbm, win2_hbm,
             wout2_hbm, out_ref, x_vmem, win_vmem, wout_vmem,
             comm_ref, rs_ref, in_sems, send_sems, recv_sems):
        my = lax.axis_index("i")

        in_copies = []
        for i, (src, dst) in enumerate([
            (x_hbm, x_vmem),
            (win0_hbm, win_vmem.at[0]), (wout0_hbm, wout_vmem.at[0]),
            (win1_hbm, win_vmem.at[1]), (wout1_hbm, wout_vmem.at[1]),
            (win2_hbm, win_vmem.at[2]), (wout2_hbm, wout_vmem.at[2]),
        ]):
            cp = pltpu.make_async_copy(src, dst, in_sems.at[i])
            cp.start()
            in_copies.append(cp)

        barrier_sem = pltpu.get_barrier_semaphore()
        for d in range(1, N_DEV):
            pl.semaphore_signal(
                barrier_sem, inc=1,
                device_id=((my + d) % N_DEV,),
                device_id_type=pl.DeviceIdType.MESH,
            )
        pl.semaphore_wait(barrier_sem, N_DEV - 1)

        in_copies[0].wait()
        pending_sends = []
        x_cur = x_vmem[:, :].astype(jnp.bfloat16)
        for l in range(N_LAYERS - 1):
            in_copies[2 * l + 1].wait()
            h = jnp.dot(x_cur, win_vmem[l, :, :].astype(jnp.bfloat16),
                        preferred_element_type=jnp.float32)
            h = jnp.maximum(h, 0.0).astype(jnp.bfloat16)
            in_copies[2 * l + 2].wait()
            partial = jnp.dot(h, wout_vmem[l, :, :].astype(jnp.bfloat16),
                              preferred_element_type=jnp.float32
                              ).astype(jnp.bfloat16)
            comm_ref[l, 0, :, :] = partial
            rdmas = []
            for d in () if NO_COMM else (2, 1, 3):
                rdma = pltpu.make_async_remote_copy(
                    src_ref=comm_ref.at[l, 0],
                    dst_ref=comm_ref.at[l, d],
                    send_sem=send_sems.at[l, d - 1],
                    recv_sem=recv_sems.at[l, d - 1],
                    device_id=((my + d) % N_DEV,),
                    device_id_type=pl.DeviceIdType.MESH,
                )
                rdma.start()
                rdmas.append(rdma)
            for rdma in rdmas:
                rdma.wait_recv()
            pending_sends += rdmas
            total = (comm_ref[l, 0, :, :].astype(jnp.float32)
                     + comm_ref[l, 1, :, :].astype(jnp.float32)
                     + comm_ref[l, 2, :, :].astype(jnp.float32)
                     + comm_ref[l, 3, :, :].astype(jnp.float32))
            x_cur = total.astype(jnp.bfloat16)

        l = N_LAYERS - 1
        in_copies[2 * l + 1].wait()
        h = jnp.dot(x_cur, win_vmem[l, :, :].astype(jnp.bfloat16),
                    preferred_element_type=jnp.float32)
        h = jnp.maximum(h, 0.0).astype(jnp.bfloat16)
        in_copies[2 * l + 2].wait()
        partial = jnp.dot(h, wout_vmem[l, :, :].astype(jnp.bfloat16),
                          preferred_element_type=jnp.float32
                          ).astype(jnp.bfloat16)
        rs_ref[0, :, :, :] = partial.reshape(N_DEV, rows_per, D)
        for d in () if NO_COMM else (2, 1, 3):
            for k in range(N_DEV):
                @pl.when(my == (k - d) % N_DEV)
                def _(k=k, d=d):
                    rdma = pltpu.make_async_remote_copy(
                        src_ref=rs_ref.at[0, k],
                        dst_ref=rs_ref.at[d, 0],
                        send_sem=send_sems.at[l, d - 1],
                        recv_sem=recv_sems.at[l, d - 1],
                        device_id=(k,),
                        device_id_type=pl.DeviceIdType.MESH,
                    )
                    rdma.start()
        for d in () if NO_COMM else (2, 1, 3):
            rdma = pltpu.make_async_remote_copy(
                src_ref=rs_ref.at[0, 0],
                dst_ref=rs_ref.at[d, 0],
                send_sem=send_sems.at[l, d - 1],
                recv_sem=recv_sems.at[l, d - 1],
                device_id=((my + d) % N_DEV,),
                device_id_type=pl.DeviceIdType.MESH,
            )
            rdma.wait_recv()
            pending_sends.append(rdma)

        for k in range(N_DEV):
            @pl.when(my == k)
            def _(k=k):
                out_ref[:, :] = (rs_ref[0, k, :, :].astype(jnp.float32)
                                 + rs_ref[1, 0, :, :].astype(jnp.float32)
                                 + rs_ref[2, 0, :, :].astype(jnp.float32)
                                 + rs_ref[3, 0, :, :].astype(jnp.float32))

        for rdma in pending_sends:
            rdma.wait_send()

    return pl.pallas_call(
        body,
        out_shape=jax.ShapeDtypeStruct((rows_per, D), jnp.float32),
        in_specs=[pl.BlockSpec(memory_space=pl.ANY)] * 7,
        out_specs=pl.BlockSpec(memory_space=pltpu.VMEM),
        scratch_shapes=[
            pltpu.VMEM((B, D), jnp.float32),
            pltpu.VMEM((N_LAYERS, D, H), jnp.float32),
            pltpu.VMEM((N_LAYERS, H, D), jnp.float32),
            pltpu.VMEM((N_LAYERS - 1, N_DEV, B, D), jnp.bfloat16),
            pltpu.VMEM((N_DEV, N_DEV, rows_per, D), jnp.bfloat16),
            pltpu.SemaphoreType.DMA((7,)),
            pltpu.SemaphoreType.DMA((N_LAYERS, N_DEV - 1)),
            pltpu.SemaphoreType.DMA((N_LAYERS, N_DEV - 1)),
        ],
        compiler_params=pltpu.CompilerParams(collective_id=0),
    )(x, Win0, Wout0, Win1, Wout1, Win2, Wout2)


# device time: 15155 ns/iter; 1.3059x vs baseline; 1.3059x over previous
import os

import jax
import jax.numpy as jnp
from jax import lax
from jax.experimental import pallas as pl
from jax.experimental.pallas import tpu as pltpu

N_DEV = 4
N_LAYERS = 3
NO_COMM = os.path.exists(os.path.join(os.path.dirname(__file__), "NO_COMM"))


def kernel(x, Win0, Wout0, Win1, Wout1, Win2, Wout2):
    B, D = x.shape
    rows_per = B // N_DEV

    parts = [x]
    for Win, Wout in ((Win0, Wout0), (Win1, Wout1), (Win2, Wout2)):
        parts += [Win[:, :D], Win[:, D:], Wout]
    packed = jnp.concatenate(parts, axis=0).astype(jnp.bfloat16)

    def body(p_ref, out_ref, comm_ref, rs_ref, send_sems, recv_sems):
        my = lax.axis_index("i")

        barrier_sem = pltpu.get_barrier_semaphore()
        for d in range(1, N_DEV):
            pl.semaphore_signal(
                barrier_sem, inc=1,
                device_id=((my + d) % N_DEV,),
                device_id_type=pl.DeviceIdType.MESH,
            )
        pl.semaphore_wait(barrier_sem, N_DEV - 1)

        def layer(l, x_cur):
            base = B + 4 * D * l
            wh0 = p_ref[base:base + D, :]
            wh1 = p_ref[base + D:base + 2 * D, :]
            wout_top = p_ref[base + 2 * D:base + 3 * D, :]
            wout_bot = p_ref[base + 3 * D:base + 4 * D, :]
            h0 = jnp.maximum(jnp.dot(x_cur, wh0,
                                     preferred_element_type=jnp.float32),
                             0.0).astype(jnp.bfloat16)
            h1 = jnp.maximum(jnp.dot(x_cur, wh1,
                                     preferred_element_type=jnp.float32),
                             0.0).astype(jnp.bfloat16)
            partial = (jnp.dot(h0, wout_top, preferred_element_type=jnp.float32)
                       + jnp.dot(h1, wout_bot, preferred_element_type=jnp.float32))
            return partial.astype(jnp.bfloat16)

        pending_sends = []
        x_cur = p_ref[0:B, :]
        for l in range(N_LAYERS - 1):
            comm_ref[l, 0, :, :] = layer(l, x_cur)
            rdmas = []
            for d in () if NO_COMM else (2, 1, 3):
                rdma = pltpu.make_async_remote_copy(
                    src_ref=comm_ref.at[l, 0],
                    dst_ref=comm_ref.at[l, d],
                    send_sem=send_sems.at[l, d - 1],
                    recv_sem=recv_sems.at[l, d - 1],
                    device_id=((my + d) % N_DEV,),
                    device_id_type=pl.DeviceIdType.MESH,
                )
                rdma.start()
                rdmas.append(rdma)
            for rdma in rdmas:
                rdma.wait_recv()
            pending_sends += rdmas
            total = (comm_ref[l, 0, :, :].astype(jnp.float32)
                     + comm_ref[l, 1, :, :].astype(jnp.float32)
                     + comm_ref[l, 2, :, :].astype(jnp.float32)
                     + comm_ref[l, 3, :, :].astype(jnp.float32))
            x_cur = total.astype(jnp.bfloat16)

        l = N_LAYERS - 1
        partial = layer(l, x_cur)
        rs_ref[0, :, :, :] = partial.reshape(N_DEV, rows_per, D)
        for d in () if NO_COMM else (2, 1, 3):
            for k in range(N_DEV):
                @pl.when(my == (k - d) % N_DEV)
                def _(k=k, d=d):
                    rdma = pltpu.make_async_remote_copy(
                        src_ref=rs_ref.at[0, k],
                        dst_ref=rs_ref.at[d, 0],
                        send_sem=send_sems.at[l, d - 1],
                        recv_sem=recv_sems.at[l, d - 1],
                        device_id=(k,),
                        device_id_type=pl.DeviceIdType.MESH,
                    )
                    rdma.start()
        for d in () if NO_COMM else (2, 1, 3):
            rdma = pltpu.make_async_remote_copy(
                src_ref=rs_ref.at[0, 0],
                dst_ref=rs_ref.at[d, 0],
                send_sem=send_sems.at[l, d - 1],
                recv_sem=recv_sems.at[l, d - 1],
                device_id=((my + d) % N_DEV,),
                device_id_type=pl.DeviceIdType.MESH,
            )
            rdma.wait_recv()
            pending_sends.append(rdma)

        for k in range(N_DEV):
            @pl.when(my == k)
            def _(k=k):
                out_ref[:, :] = (rs_ref[0, k, :, :].astype(jnp.float32)
                                 + rs_ref[1, 0, :, :].astype(jnp.float32)
                                 + rs_ref[2, 0, :, :].astype(jnp.float32)
                                 + rs_ref[3, 0, :, :].astype(jnp.float32))

        for rdma in pending_sends:
            rdma.wait_send()

    return pl.pallas_call(
        body,
        out_shape=jax.ShapeDtypeStruct((rows_per, D), jnp.float32),
        in_specs=[pl.BlockSpec(memory_space=pltpu.VMEM)],
        out_specs=pl.BlockSpec(memory_space=pltpu.VMEM),
        scratch_shapes=[
            pltpu.VMEM((N_LAYERS - 1, N_DEV, B, D), jnp.bfloat16),
            pltpu.VMEM((N_DEV, N_DEV, rows_per, D), jnp.bfloat16),
            pltpu.SemaphoreType.DMA((N_LAYERS, N_DEV - 1)),
            pltpu.SemaphoreType.DMA((N_LAYERS, N_DEV - 1)),
        ],
        compiler_params=pltpu.CompilerParams(collective_id=0),
    )(packed)
